# baseline (speedup 1.0000x reference)
import functools

import jax
import jax.numpy as jnp
from jax import lax
from jax.experimental import pallas as pl
from jax.experimental.pallas import tpu as pltpu
from jax.experimental.pallas import tpu_sc as plsc

ESIZE = 64
NBUF = 8

_info = plsc.get_sparse_core_info()
NC, NS = _info.num_cores, _info.num_subcores
NW = NC * NS
L = 16


@jax.jit
def _format_table(embT, tail128):
    _, v = embT.shape
    nslab = v // 128
    vmain = nslab * 128
    vpad = vmain + 128
    mesh = plsc.VectorSubcoreMesh(core_axis_name="c", subcore_axis_name="s")

    @functools.partial(
        pl.kernel,
        out_type=jax.ShapeDtypeStruct((vpad // 2, 128), jnp.float32),
        mesh=mesh,
        scratch_types=[
            pltpu.VMEM((2, ESIZE, 128), jnp.float32),
            pltpu.VMEM((ESIZE, 128), jnp.float32),
            pltpu.VMEM((ESIZE * 128,), jnp.int32),
            pltpu.VMEM((32, 128), jnp.float32),
            pltpu.SemaphoreType.DMA((2,)),
        ],
        compiler_params=pltpu.CompilerParams(
            use_tc_tiling_on_sc=True, needs_layout_passes=False),
    )
    def k(embT_hbm, tail_hbm, out_hbm, buf_a, buf_b, idx_t, tail_v, rsem):
        wid = lax.axis_index("s") * NC + lax.axis_index("c")
        nw_slabs = 244 + (nslab - 244 * NW > wid).astype(jnp.int32)

        def init(t, _):
            w = t * L + lax.iota(jnp.int32, L)
            p = w >> 7
            c = w & 127
            a = ((c & 63) << 7) + 2 * p + (c >> 6)
            idx_t[pl.ds(t * L, L)] = a
            return _

        lax.fori_loop(0, ESIZE * 128 // L, init, None)

        def slab_of(i):
            return wid + i * NW

        def read(i, b):
            return pltpu.make_async_copy(
                embT_hbm.at[:, pl.ds(slab_of(i) * 128, 128)], buf_a.at[b],
                rsem.at[b])

        read(0, 0).start()

        def body(i, _):
            b = i % 2
            read(i, b).wait()

            @pl.when(i + 1 < nw_slabs)
            def _prefetch():
                read(i + 1, 1 - b).start()

            def transpose(t, _):
                a = idx_t[pl.ds(t * L, L)]
                vals = plsc.load_gather(buf_a.at[b], [a >> 7, a & 127])
                buf_b[t >> 3, pl.ds((t & 7) * L, L)] = vals
                return _

            lax.fori_loop(0, ESIZE * 128 // L, transpose, None)
            pltpu.sync_copy(buf_b, out_hbm.at[pl.ds(slab_of(i) * 64, 64)])
            return _

        lax.fori_loop(0, nw_slabs, body, None)

        @pl.when(wid == NW - 1)
        def _tail():
            pltpu.sync_copy(tail_hbm, tail_v)
            pltpu.sync_copy(tail_v, out_hbm.at[pl.ds(vmain // 2, 32)])

    return k(embT, tail128)


@jax.jit
def _gather_rows(x_eo, tab):
    nb, _, nsh = x_eo.shape
    xrows_per_w = nb // NW
    ngrp = xrows_per_w // NBUF
    assert ngrp * NBUF * NW == nb
    mesh = plsc.VectorSubcoreMesh(core_axis_name="c", subcore_axis_name="s")

    @functools.partial(
        pl.kernel,
        out_type=jax.ShapeDtypeStruct((nb * nsh, 2 * ESIZE), jnp.float32),
        mesh=mesh,
        scratch_types=[
            pltpu.VMEM((xrows_per_w, 2, nsh), jnp.int32),
            pltpu.VMEM((NBUF, nsh, ESIZE), jnp.float32),
            pltpu.VMEM((NBUF, nsh, ESIZE), jnp.float32),
            pltpu.SemaphoreType.DMA((NBUF,)),
            pltpu.SemaphoreType.DMA((NBUF,)),
        ],
        compiler_params=pltpu.CompilerParams(use_tc_tiling_on_sc=False),
    )
    def k(tab_hbm, x_hbm, out_hbm, idx_v, rows_a, rows_b, gsem, ssem):
        wid = lax.axis_index("s") * NC + lax.axis_index("c")
        base = wid * xrows_per_w
        pltpu.sync_copy(x_hbm.at[pl.ds(base, xrows_per_w)], idx_v)

        def gathers(r, b):
            return (
                pltpu.make_async_copy(
                    tab_hbm.at[idx_v.at[r, 0]], rows_a.at[b], gsem.at[b]),
                pltpu.make_async_copy(
                    tab_hbm.at[idx_v.at[r, 1]], rows_b.at[b], gsem.at[b]),
            )

        def stores(r, b):
            p0 = (base + r) * nsh
            return (
                pltpu.make_async_copy(
                    rows_a.at[b], out_hbm.at[pl.ds(p0, nsh), pl.ds(0, ESIZE)],
                    ssem.at[b]),
                pltpu.make_async_copy(
                    rows_b.at[b], out_hbm.at[pl.ds(p0, nsh), pl.ds(ESIZE, ESIZE)],
                    ssem.at[b]),
            )

        def start(descs):
            for d in descs:
                d.start()

        def wait(descs):
            for d in descs:
                d.wait()

        for b in range(NBUF):
            start(gathers(b, b))

        def group(g, _):
            r0 = g * NBUF
            for b in range(NBUF):
                r = r0 + b
                wait(gathers(r, b))
                start(stores(r, b))
                wait(stores(r, b))
                start(gathers(r + NBUF, b))
            return _

        lax.fori_loop(0, ngrp - 1, group, None)

        r0 = (ngrp - 1) * NBUF
        for b in range(NBUF):
            r = r0 + b
            wait(gathers(r, b))
            start(stores(r, b))
            wait(stores(r, b))

    return k(tab, x_eo)


def kernel(x, emb):
    nb, nseq = x.shape
    v, esize = emb.shape
    vmain = (v // 128) * 128
    xi = x.astype(jnp.int32)
    x_eo = jnp.stack([xi[:, 0::2], xi[:, 1::2]], axis=1)
    tail128 = jnp.concatenate([emb[vmain::2, :], emb[vmain + 1::2, :]], axis=1)
    tabp = _format_table(emb.T, tail128)
    tab = tabp.reshape(-1, esize)
    out = _gather_rows(x_eo, tab)
    return out.reshape(nb, nseq, esize)


# speedup vs baseline: 2.2217x; 2.2217x over previous
import functools

import jax
import jax.numpy as jnp
from jax import lax
from jax.experimental import pallas as pl
from jax.experimental.pallas import tpu as pltpu
from jax.experimental.pallas import tpu_sc as plsc

ESIZE = 64
NBUF = 8

_info = plsc.get_sparse_core_info()
NC, NS = _info.num_cores, _info.num_subcores
NW = NC * NS


@jax.jit
def _gather_rows(x, emb):
    nb, nseq = x.shape
    assert nseq <= 128
    xrows_per_w = nb // NW
    ngrp = xrows_per_w // NBUF
    assert ngrp * NBUF * NW == nb
    mesh = plsc.VectorSubcoreMesh(core_axis_name="c", subcore_axis_name="s")

    @functools.partial(
        pl.kernel,
        out_type=jax.ShapeDtypeStruct((nb, nseq, ESIZE), jnp.float32),
        mesh=mesh,
        scratch_types=[
            pltpu.VMEM((xrows_per_w, nseq), jnp.int32),
            pltpu.VMEM((NBUF, nseq, ESIZE), jnp.float32),
            pltpu.SemaphoreType.DMA((NBUF,)),
            pltpu.SemaphoreType.DMA((NBUF,)),
        ],
        compiler_params=pltpu.CompilerParams(use_tc_tiling_on_sc=False),
    )
    def k(emb_hbm, x_hbm, out_hbm, idx_v, rows_v, gsem, ssem):
        wid = lax.axis_index("s") * NC + lax.axis_index("c")
        base = wid * xrows_per_w
        pltpu.sync_copy(x_hbm.at[pl.ds(base, xrows_per_w)], idx_v)

        def gather(r, b):
            return pltpu.make_async_copy(
                emb_hbm.at[idx_v.at[r]], rows_v.at[b], gsem.at[b]
            )

        def store(r, b):
            return pltpu.make_async_copy(
                rows_v.at[b], out_hbm.at[base + r], ssem.at[b]
            )

        for b in range(NBUF):
            gather(b, b).start()

        def group(g, _):
            r0 = g * NBUF
            for b in range(NBUF):
                r = r0 + b
                gather(r, b).wait()
                store(r, b).start()
                store(r, b).wait()
                gather(r + NBUF, b).start()
            return _

        lax.fori_loop(0, ngrp - 1, group, None)

        r0 = (ngrp - 1) * NBUF
        for b in range(NBUF):
            r = r0 + b
            gather(r, b).wait()
            store(r, b).start()
            store(r, b).wait()

    return k(emb, x)


def kernel(x, emb):
    return _gather_rows(x.astype(jnp.int32), emb)
